# baseline (speedup 1.0000x reference)
import functools

import jax
import jax.numpy as jnp
from jax import lax
from jax.experimental import pallas as pl
from jax.experimental.pallas import tpu as pltpu
from jax.experimental.pallas import tpu_sc as plsc

EMBED = 64
VOCAB = 100000
BT = 4096 * 50
NC, NS = 2, 16
NW = NC * NS
PER_W = BT // NW
CHUNK = 64
N_CHUNKS = PER_W // CHUNK
NBUF = 5
LOOK = 3
GROUPS = N_CHUNKS // NBUF

CMB_R = 2000


def _combine(src_table, tgt_table):
    def body(s_ref, t_ref, o_ref):
        o_ref[:, :EMBED] = s_ref[...]
        o_ref[:, EMBED:] = t_ref[...]

    return pl.pallas_call(
        body,
        grid=(VOCAB // CMB_R,),
        in_specs=[
            pl.BlockSpec((CMB_R, EMBED), lambda i: (i, 0)),
            pl.BlockSpec((CMB_R, EMBED), lambda i: (i, 0)),
        ],
        out_specs=pl.BlockSpec((CMB_R, 2 * EMBED), lambda i: (i, 0)),
        out_shape=jax.ShapeDtypeStruct((VOCAB, 2 * EMBED), jnp.float32),
    )(src_table, tgt_table)


def _make_kernel():
    mesh = plsc.VectorSubcoreMesh(core_axis_name="c", subcore_axis_name="s")

    @functools.partial(
        pl.kernel,
        mesh=mesh,
        out_type=(
            jax.ShapeDtypeStruct((BT, EMBED), jnp.float32),
            jax.ShapeDtypeStruct((BT, EMBED), jnp.float32),
        ),
        scratch_types=[
            pltpu.VMEM((PER_W,), jnp.int32),
            pltpu.VMEM((PER_W,), jnp.int32),
            pltpu.VMEM((NBUF, CHUNK, 2 * EMBED), jnp.float32),
            pltpu.VMEM((NBUF, CHUNK, 2 * EMBED), jnp.float32),
            pltpu.SemaphoreType.DMA((NBUF,)),
            pltpu.SemaphoreType.DMA((NBUF,)),
            pltpu.SemaphoreType.DMA((NBUF,)),
            pltpu.SemaphoreType.DMA((NBUF,)),
        ],
        compiler_params=pltpu.CompilerParams(use_tc_tiling_on_sc=False),
    )
    def k(src_idx, tgt_idx, combined, src_out, tgt_out,
          idx_s, idx_t, rows_s, rows_t, gsem_s, ssem_s, gsem_t, ssem_t):
        wid = lax.axis_index("s") * NC + lax.axis_index("c")
        base_w = wid * PER_W

        pltpu.sync_copy(src_idx.at[pl.ds(base_w, PER_W)], idx_s)
        pltpu.sync_copy(tgt_idx.at[pl.ds(base_w, PER_W)], idx_t)

        streams = (
            (idx_s, src_out, rows_s, gsem_s, ssem_s, 0),
            (idx_t, tgt_out, rows_t, gsem_t, ssem_t, EMBED),
        )

        def fire_gather(st, t, b):
            idx, _, rows, gsem, _, _ = st
            pltpu.make_async_copy(
                combined.at[idx.at[pl.ds(t * CHUNK, CHUNK)]],
                rows.at[b], gsem.at[b]).start()

        def wait_gather(st, t, b):
            idx, _, rows, gsem, _, _ = st
            pltpu.make_async_copy(
                combined.at[idx.at[pl.ds(t * CHUNK, CHUNK)]],
                rows.at[b], gsem.at[b]).wait()

        def fire_store(st, t, b):
            _, out, rows, _, ssem, lo = st
            pltpu.make_async_copy(
                rows.at[b, :, pl.ds(lo, EMBED)],
                out.at[pl.ds(base_w + t * CHUNK, CHUNK)],
                ssem.at[b]).start()

        def wait_store(st, t, b):
            _, out, rows, _, ssem, lo = st
            pltpu.make_async_copy(
                rows.at[b, :, pl.ds(lo, EMBED)],
                out.at[pl.ds(base_w + t * CHUNK, CHUNK)],
                ssem.at[b]).wait()

        for b in range(LOOK):
            for st in streams:
                fire_gather(st, b, b)

        def body(g, carry):
            for j in range(NBUF):
                t = g * NBUF + j
                bn = (j + LOOK) % NBUF
                tn = t + LOOK
                for st in streams:
                    @pl.when(tn < N_CHUNKS)
                    def _():
                        @pl.when(tn >= NBUF)
                        def _():
                            wait_store(st, tn - NBUF, bn)
                        fire_gather(st, tn, bn)
                    wait_gather(st, t, j)
                    fire_store(st, t, j)
            return carry

        lax.fori_loop(0, GROUPS, body, 0)

        for kk in range(NBUF):
            t = N_CHUNKS - NBUF + kk
            for st in streams:
                wait_store(st, t, t % NBUF)

    return k


_lookup = _make_kernel()


def kernel(src, tgt, src_table, tgt_table):
    B, L = src.shape
    E = src_table.shape[1]
    src_flat = src.reshape(-1).astype(jnp.int32)
    tgt_flat = tgt.reshape(-1).astype(jnp.int32)
    src_out, tgt_out = _lookup(
        src_flat, tgt_flat, _combine(src_table, tgt_table))
    return (src_out.reshape(B, L, E), tgt_out.reshape(B, L, E))


# speedup vs baseline: 1.1479x; 1.1479x over previous
import functools

import jax
import jax.numpy as jnp
from jax import lax
from jax.experimental import pallas as pl
from jax.experimental.pallas import tpu as pltpu
from jax.experimental.pallas import tpu_sc as plsc

EMBED = 64
BT = 4096 * 50
NC, NS = 2, 16
NW = NC * NS
PER_W = BT // NW
CHUNK = 200
N_CHUNKS = PER_W // CHUNK
NBUF = 4
LOOK = 3
GROUPS = N_CHUNKS // NBUF


def _make_kernel():
    mesh = plsc.VectorSubcoreMesh(core_axis_name="c", subcore_axis_name="s")

    @functools.partial(
        pl.kernel,
        mesh=mesh,
        out_type=(
            jax.ShapeDtypeStruct((BT, EMBED), jnp.float32),
            jax.ShapeDtypeStruct((BT, EMBED), jnp.float32),
        ),
        scratch_types=[
            pltpu.VMEM((PER_W,), jnp.int32),
            pltpu.VMEM((PER_W,), jnp.int32),
            pltpu.VMEM((NBUF, CHUNK, EMBED), jnp.float32),
            pltpu.VMEM((NBUF, CHUNK, EMBED), jnp.float32),
            pltpu.SemaphoreType.DMA((NBUF,)),
            pltpu.SemaphoreType.DMA((NBUF,)),
            pltpu.SemaphoreType.DMA((NBUF,)),
            pltpu.SemaphoreType.DMA((NBUF,)),
        ],
        compiler_params=pltpu.CompilerParams(use_tc_tiling_on_sc=False),
    )
    def k(src_idx, tgt_idx, src_table, tgt_table, src_out, tgt_out,
          idx_s, idx_t, rows_s, rows_t, gsem_s, ssem_s, gsem_t, ssem_t):
        wid = lax.axis_index("s") * NC + lax.axis_index("c")
        base_w = wid * PER_W

        pltpu.sync_copy(src_idx.at[pl.ds(base_w, PER_W)], idx_s)
        pltpu.sync_copy(tgt_idx.at[pl.ds(base_w, PER_W)], idx_t)

        streams = (
            (idx_s, src_table, src_out, rows_s, gsem_s, ssem_s),
            (idx_t, tgt_table, tgt_out, rows_t, gsem_t, ssem_t),
        )

        def fire_gather(st, t, b):
            idx, table, _, rows, gsem, _ = st
            pltpu.make_async_copy(
                table.at[idx.at[pl.ds(t * CHUNK, CHUNK)]],
                rows.at[b], gsem.at[b]).start()

        def wait_gather(st, t, b):
            idx, table, _, rows, gsem, _ = st
            pltpu.make_async_copy(
                table.at[idx.at[pl.ds(t * CHUNK, CHUNK)]],
                rows.at[b], gsem.at[b]).wait()

        def fire_store(st, t, b):
            _, _, out, rows, _, ssem = st
            pltpu.make_async_copy(
                rows.at[b], out.at[pl.ds(base_w + t * CHUNK, CHUNK)],
                ssem.at[b]).start()

        def wait_store(st, t, b):
            _, _, out, rows, _, ssem = st
            pltpu.make_async_copy(
                rows.at[b], out.at[pl.ds(base_w + t * CHUNK, CHUNK)],
                ssem.at[b]).wait()

        for b in range(LOOK):
            for st in streams:
                fire_gather(st, b, b)

        def body(g, carry):
            for j in range(NBUF):
                t = g * NBUF + j
                bn = (j + LOOK) % NBUF
                tn = t + LOOK
                for st in streams:
                    @pl.when(tn < N_CHUNKS)
                    def _():
                        @pl.when(tn >= NBUF)
                        def _():
                            wait_store(st, tn - NBUF, bn)
                        fire_gather(st, tn, bn)
                    wait_gather(st, t, j)
                    fire_store(st, t, j)
            return carry

        lax.fori_loop(0, GROUPS, body, 0)

        for kk in range(NBUF):
            t = N_CHUNKS - NBUF + kk
            for st in streams:
                wait_store(st, t, t % NBUF)

    return k


_lookup = _make_kernel()


def kernel(src, tgt, src_table, tgt_table):
    B, L = src.shape
    E = src_table.shape[1]
    src_flat = src.reshape(-1).astype(jnp.int32)
    tgt_flat = tgt.reshape(-1).astype(jnp.int32)
    src_out, tgt_out = _lookup(src_flat, tgt_flat, src_table, tgt_table)
    return (src_out.reshape(B, L, E), tgt_out.reshape(B, L, E))
